# baseline (speedup 1.0000x reference)
import jax
import jax.numpy as jnp
from jax import lax
from jax.experimental import pallas as pl
from jax.experimental.pallas import tpu as pltpu


def _round_up(x: int, m: int) -> int:
    return ((x + m - 1) // m) * m


_GROUP_UNROLL = 2


def _gather_rows_kernel(base8_ref, shift_ref, bank_ref, out_ref):
    tile_b, lat_p = out_ref.shape
    base = pl.program_id(0) * tile_b
    sublane = lax.broadcasted_iota(jnp.int32, (8, lat_p), 0)
    m_odd = (sublane & 1) == 1
    m_23 = (sublane & 2) == 2
    m_hi = sublane >= 4

    def group_body(g, carry):
        for u in range(_GROUP_UNROLL):
            row0 = base + (g * _GROUP_UNROLL + u) * 8
            rolled = []
            for s in range(8):
                cb = pl.multiple_of(base8_ref[row0 + s], 8)
                chunk = bank_ref[pl.ds(cb, 8), :]
                rolled.append(pltpu.roll(chunk, shift_ref[row0 + s], axis=0))
            t01 = jnp.where(m_odd, rolled[1], rolled[0])
            t23 = jnp.where(m_odd, rolled[3], rolled[2])
            t45 = jnp.where(m_odd, rolled[5], rolled[4])
            t67 = jnp.where(m_odd, rolled[7], rolled[6])
            t03 = jnp.where(m_23, t23, t01)
            t47 = jnp.where(m_23, t67, t45)
            acc = jnp.where(m_hi, t47, t03)
            out_ref[pl.ds(pl.multiple_of(row0 - base, 8), 8), :] = acc
        return carry

    lax.fori_loop(0, tile_b // (8 * _GROUP_UNROLL), group_body, 0)


def _run_gather(base8, shift, bank_p, tile_b):
    padded = base8.shape[0]
    num_styles, lat_p = bank_p.shape
    grid = (padded // tile_b,)
    itemsize = jnp.dtype(bank_p.dtype).itemsize
    bank_bytes = num_styles * lat_p * itemsize
    cost = pl.CostEstimate(
        flops=0, transcendentals=0,
        bytes_accessed=int(bank_bytes + padded * (lat_p * itemsize + 8)))
    vmem_limit = int(min(bank_bytes + 2 * tile_b * lat_p * itemsize + (4 << 20),
                         48 << 20))

    return pl.pallas_call(
        _gather_rows_kernel,
        out_shape=jax.ShapeDtypeStruct((padded, lat_p), bank_p.dtype),
        grid_spec=pltpu.PrefetchScalarGridSpec(
            num_scalar_prefetch=2,
            grid=grid,
            in_specs=[
                pl.BlockSpec((num_styles, lat_p), lambda i, b, s: (0, 0),
                             pipeline_mode=pl.Buffered(1)),
            ],
            out_specs=pl.BlockSpec((tile_b, lat_p), lambda i, b, s: (i, 0)),
        ),
        compiler_params=pltpu.CompilerParams(
            dimension_semantics=("arbitrary",),
            vmem_limit_bytes=vmem_limit,
        ),
        cost_estimate=cost,
    )(base8, shift, bank_p)


def kernel(style_bank, writer_id):
    num_styles, latent_dim = style_bank.shape
    orig_shape = jnp.shape(writer_id)

    wid = jnp.asarray(writer_id, jnp.int32).reshape(-1)
    batch = wid.shape[0]

    lat_p = _round_up(latent_dim, 128)
    bank_p = style_bank if lat_p == latent_dim else jnp.pad(
        style_bank, ((0, 0), (0, lat_p - latent_dim)))

    group = 8 * _GROUP_UNROLL
    tile_b = 1024
    while tile_b > group and batch < tile_b:
        tile_b //= 2
    padded = _round_up(batch, tile_b)
    if padded != batch:
        wid = jnp.pad(wid, (0, padded - batch))

    wid = jnp.clip(wid, 0, num_styles - 1)
    base8 = (wid >> 3) << 3
    pos = lax.iota(jnp.int32, padded)
    shift = ((pos & 7) - (wid & 7)) & 7

    out_flat = _run_gather(base8, shift, bank_p, tile_b)
    out_flat = out_flat[:batch, :latent_dim]
    return out_flat.reshape(orig_shape + (latent_dim,))


# speedup vs baseline: 1.0168x; 1.0168x over previous
import jax
import jax.numpy as jnp
from jax import lax
from jax.experimental import pallas as pl
from jax.experimental.pallas import tpu as pltpu


def _round_up(x: int, m: int) -> int:
    return ((x + m - 1) // m) * m


_GROUP_UNROLL = 8


def _gather_rows_kernel(base8_ref, shift_ref, bank_ref, out_ref):
    tile_b, lat_p = out_ref.shape
    base = pl.program_id(0) * tile_b
    sublane = lax.broadcasted_iota(jnp.int32, (8, lat_p), 0)
    masks = [sublane == s for s in range(1, 8)]

    def group_body(g, carry):
        for u in range(_GROUP_UNROLL):
            row0 = base + (g * _GROUP_UNROLL + u) * 8
            acc = None
            for s in range(8):
                cb = pl.multiple_of(base8_ref[row0 + s], 8)
                chunk = bank_ref[pl.ds(cb, 8), :]
                rolled = pltpu.roll(chunk, shift_ref[row0 + s], axis=0)
                acc = rolled if s == 0 else jnp.where(masks[s - 1], rolled, acc)
            out_ref[pl.ds(pl.multiple_of(row0 - base, 8), 8), :] = acc
        return carry

    lax.fori_loop(0, tile_b // (8 * _GROUP_UNROLL), group_body, 0)


def _run_gather(base8, shift, bank_p, tile_b):
    padded = base8.shape[0]
    num_styles, lat_p = bank_p.shape
    grid = (padded // tile_b,)
    itemsize = jnp.dtype(bank_p.dtype).itemsize
    bank_bytes = num_styles * lat_p * itemsize
    cost = pl.CostEstimate(
        flops=0, transcendentals=0,
        bytes_accessed=int(bank_bytes + padded * (lat_p * itemsize + 8)))
    vmem_limit = int(min(bank_bytes + 2 * tile_b * lat_p * itemsize + (4 << 20),
                         48 << 20))

    return pl.pallas_call(
        _gather_rows_kernel,
        out_shape=jax.ShapeDtypeStruct((padded, lat_p), bank_p.dtype),
        grid_spec=pltpu.PrefetchScalarGridSpec(
            num_scalar_prefetch=2,
            grid=grid,
            in_specs=[
                pl.BlockSpec((num_styles, lat_p), lambda i, b, s: (0, 0),
                             pipeline_mode=pl.Buffered(1)),
            ],
            out_specs=pl.BlockSpec((tile_b, lat_p), lambda i, b, s: (i, 0)),
        ),
        compiler_params=pltpu.CompilerParams(
            dimension_semantics=("arbitrary",),
            vmem_limit_bytes=vmem_limit,
        ),
        cost_estimate=cost,
    )(base8, shift, bank_p)


def kernel(style_bank, writer_id):
    num_styles, latent_dim = style_bank.shape
    orig_shape = jnp.shape(writer_id)

    wid = jnp.asarray(writer_id, jnp.int32).reshape(-1)
    batch = wid.shape[0]

    lat_p = _round_up(latent_dim, 128)
    bank_p = style_bank if lat_p == latent_dim else jnp.pad(
        style_bank, ((0, 0), (0, lat_p - latent_dim)))

    group = 8 * _GROUP_UNROLL
    tile_b = 512
    while tile_b > group and batch < tile_b:
        tile_b //= 2
    padded = _round_up(batch, tile_b)
    if padded != batch:
        wid = jnp.pad(wid, (0, padded - batch))

    wid = jnp.clip(wid, 0, num_styles - 1)
    base8 = (wid >> 3) << 3
    pos = lax.iota(jnp.int32, padded)
    shift = ((pos & 7) - (wid & 7)) & 7

    out_flat = _run_gather(base8, shift, bank_p, tile_b)
    out_flat = out_flat[:batch, :latent_dim]
    return out_flat.reshape(orig_shape + (latent_dim,))
